# baseline (speedup 1.0000x reference)
import functools

import jax
import jax.numpy as jnp
from jax import lax
from jax.experimental import pallas as pl
from jax.experimental.pallas import tpu as pltpu
from jax.experimental.pallas import tpu_sc as plsc

B, T, H = 1024, 200, 768
POS_OFF = 2
N = B * T
NC, NS = 2, 16
NW = NC * NS
ROWS_W = N // NW
SEQ_W = ROWS_W // T
TCH = 40
NTC = T // TCH
VPR = H // 16
NB = 2

_mesh = plsc.VectorSubcoreMesh(core_axis_name="c", subcore_axis_name="s")


@functools.partial(
    pl.kernel,
    out_type=jax.ShapeDtypeStruct((N, H), jnp.float32),
    mesh=_mesh,
    scratch_types=[
        pltpu.VMEM((ROWS_W,), jnp.int32),
        pltpu.VMEM((TCH, H), jnp.float32),
        pltpu.VMEM((NB, TCH, H), jnp.float32),
        pltpu.SemaphoreType.DMA,
        pltpu.SemaphoreType.DMA,
        pltpu.SemaphoreType.DMA,
        pltpu.SemaphoreType.DMA,
    ],
)
def _emb(x_hbm, wtab_hbm, pos_hbm, out_hbm, idx_v, pos_v, ring_v,
         sg0, sg1, so0, so1):
    sg = [sg0, sg1]
    so = [so0, so1]
    wid = lax.axis_index("s") * NC + lax.axis_index("c")
    base = wid * ROWS_W
    pltpu.sync_copy(x_hbm.at[pl.ds(base, ROWS_W)], idx_v)

    for tc in range(NTC):
        pltpu.sync_copy(pos_hbm.at[pl.ds(tc * TCH, TCH)], pos_v)

        def gather_desc(g, b):
            loc = g * T + tc * TCH
            return pltpu.make_async_copy(
                wtab_hbm.at[idx_v.at[pl.ds(loc, TCH)]], ring_v.at[b], sg[b])

        def out_desc(g, b):
            loc = g * T + tc * TCH
            return pltpu.make_async_copy(
                ring_v.at[b], out_hbm.at[pl.ds(base + loc, TCH)], so[b])

        for b in range(NB):
            gather_desc(b, b).start()

        @pl.loop(0, SEQ_W, step=NB)
        def _grp(g0):
            for b in range(NB):
                g = g0 + b
                gather_desc(g, b).wait()

                @pl.loop(0, TCH)
                def _row(r):
                    for c in range(VPR):
                        sl = pl.ds(c * 16, 16)
                        plsc.addupdate(ring_v.at[b, r, sl], pos_v[r, sl])

                out_desc(g, b).start()
                pb = (b - 1) % NB
                pg = g + NB - 1

                @pl.when(jnp.logical_and(pg >= NB, pg < SEQ_W))
                def _refill():
                    out_desc(g - 1, pb).wait()
                    gather_desc(pg, pb).start()

        for b in range(NB):
            out_desc(SEQ_W - NB + b, b).wait()


def kernel(x, word_embeddings, position_embeddings):
    xf = x.reshape(N)
    pos2 = lax.slice_in_dim(position_embeddings, POS_OFF, POS_OFF + T, axis=0)
    out = _emb(xf, word_embeddings, pos2)
    return out.reshape(B, T, H)


# speedup vs baseline: 2.1223x; 2.1223x over previous
import functools

import jax
import jax.numpy as jnp
from jax import lax
from jax.experimental import pallas as pl
from jax.experimental.pallas import tpu as pltpu
from jax.experimental.pallas import tpu_sc as plsc

B, T, H = 1024, 200, 768
POS_OFF = 2
N = B * T
NC, NS = 2, 16
NW = NC * NS
ROWS_W = N // NW
SEQ_W = ROWS_W // T
TCH = 40
NTC = T // TCH
VPR = H // 16
NB = 3

_mesh = plsc.VectorSubcoreMesh(core_axis_name="c", subcore_axis_name="s")


@functools.partial(
    pl.kernel,
    out_type=jax.ShapeDtypeStruct((N, H), jnp.float32),
    mesh=_mesh,
    scratch_types=[
        pltpu.VMEM((ROWS_W,), jnp.int32),
        pltpu.VMEM((TCH, H), jnp.float32),
        pltpu.VMEM((NB, TCH, H), jnp.float32),
        pltpu.SemaphoreType.DMA,
        pltpu.SemaphoreType.DMA,
        pltpu.SemaphoreType.DMA,
        pltpu.SemaphoreType.DMA,
        pltpu.SemaphoreType.DMA,
        pltpu.SemaphoreType.DMA,
    ],
)
def _emb(x_hbm, wtab_hbm, pos_hbm, out_hbm, idx_v, pos_v, ring_v,
         sg0, sg1, sg2, so0, so1, so2):
    sg = [sg0, sg1, sg2]
    so = [so0, so1, so2]
    wid = lax.axis_index("s") * NC + lax.axis_index("c")
    base = wid * ROWS_W
    pltpu.sync_copy(x_hbm.at[pl.ds(base, ROWS_W)], idx_v)

    @pl.loop(0, NTC)
    def _tc(tc):
        pltpu.sync_copy(pos_hbm.at[pl.ds(tc * TCH, TCH)], pos_v)

        def gather_desc(g, b):
            loc = g * T + tc * TCH
            return pltpu.make_async_copy(
                wtab_hbm.at[idx_v.at[pl.ds(loc, TCH)]], ring_v.at[b], sg[b])

        def out_desc(g, b):
            loc = g * T + tc * TCH
            return pltpu.make_async_copy(
                ring_v.at[b], out_hbm.at[pl.ds(base + loc, TCH)], so[b])

        for b in range(NB):
            gather_desc(b, b).start()

        REM = SEQ_W % NB
        MAIN = SEQ_W - REM

        def slot(g, b, refill):
            gather_desc(g, b).wait()

            @plsc.parallel_loop(0, TCH)
            def _row(r):
                for c in range(VPR):
                    sl = pl.ds(c * 16, 16)
                    plsc.addupdate(ring_v.at[b, r, sl], pos_v[r, sl])

            out_desc(g, b).start()
            if refill:
                pb = (b - 1) % NB
                pg = g + NB - 1

                @pl.when(g >= 1)
                def _refill():
                    out_desc(g - 1, pb).wait()
                    gather_desc(pg, pb).start()

        @pl.loop(0, MAIN, step=NB)
        def _grp(g0):
            for b in range(NB):
                slot(g0 + b, b, refill=True)

        for k in range(REM):
            g = MAIN + k
            slot(g, g % NB, refill=False)

        for g in range(MAIN - 1, SEQ_W):
            out_desc(g, g % NB).wait()


def kernel(x, word_embeddings, position_embeddings):
    xf = x.reshape(N)
    pos2 = lax.slice_in_dim(position_embeddings, POS_OFF, POS_OFF + T, axis=0)
    out = _emb(xf, word_embeddings, pos2)
    return out.reshape(B, T, H)
